# baseline (speedup 1.0000x reference)
import functools

import jax
import jax.numpy as jnp
from jax import lax
from jax.experimental import pallas as pl
from jax.experimental.pallas import tpu as pltpu
from jax.experimental.pallas import tpu_sc as plsc

LANES = 16
OCT = 8


@functools.cache
def _make_gather2(batch, hist, depth):
    d2 = 2 * depth
    hist2 = hist // 2
    out_rows = batch * hist2
    info = plsc.get_sparse_core_info()
    nw = info.num_cores * info.num_subcores
    entries_per_worker = batch // nw
    n_oct = entries_per_worker // OCT
    n_full = hist // LANES
    tail = hist - LANES if hist % LANES else None
    mesh = plsc.VectorSubcoreMesh(core_axis_name="c", subcore_axis_name="s")

    @functools.partial(
        pl.kernel,
        mesh=mesh,
        compiler_params=pltpu.CompilerParams(use_tc_tiling_on_sc=True),
        out_type=[
            jax.ShapeDtypeStruct((out_rows, d2), jnp.float32),
            jax.ShapeDtypeStruct((out_rows, d2), jnp.float32),
        ],
        scratch_types=[
            pltpu.VMEM((OCT, hist), jnp.int32),
            pltpu.VMEM((OCT, hist), jnp.int32),
            pltpu.VMEM((hist, d2), jnp.float32),
            pltpu.VMEM((hist, d2), jnp.float32),
            pltpu.VMEM((hist, d2), jnp.float32),
            pltpu.VMEM((hist, d2), jnp.float32),
            pltpu.SemaphoreType.DMA,
            pltpu.SemaphoreType.DMA,
            pltpu.SemaphoreType.DMA,
            pltpu.SemaphoreType.DMA,
        ],
    )
    def gather2(qi_hbm, vi_hbm, qt_hbm, kt_hbm, qo_hbm, vo_hbm,
                idx_v, half_v, rows0, rows1, comp0, comp1,
                sg0, sg1, sw0, sw1):
        wid = lax.axis_index("s") * info.num_cores + lax.axis_index("c")
        ebase = wid * entries_per_worker
        rows = (rows0, rows1)
        comp = (comp0, comp1)
        sg = (sg0, sg1)
        sw = (sw0, sw1)

        def run_table(idx_hbm, tab_hbm, out_hbm):
            def fire(e, slot):
                for kb in range(n_full):
                    sl = pl.ds(kb * LANES, LANES)
                    half_v[e, sl] = lax.shift_right_logical(idx_v[e, sl], 1)
                if tail is not None:
                    sl = pl.ds(tail, LANES)
                    half_v[e, sl] = lax.shift_right_logical(idx_v[e, sl], 1)
                return [
                    pltpu.async_copy(
                        tab_hbm.at[half_v.at[e, pl.ds(0, 128)]],
                        rows[slot].at[pl.ds(0, 128)],
                        sg[slot],
                    ),
                    pltpu.async_copy(
                        tab_hbm.at[half_v.at[e, pl.ds(128, hist - 128)]],
                        rows[slot].at[pl.ds(128, hist - 128)],
                        sg[slot],
                    ),
                ]

            def compact(e, rslot, cslot):
                rows_v, comp_v = rows[rslot], comp[cslot]
                crow0 = hist2 * (e % 2)

                def cblock(st, sthalf):
                    starts = (idx_v[e, pl.ds(st, LANES)] & 1) * depth
                    for l in range(LANES):
                        s0 = starts[l]
                        dst_r = crow0 + sthalf + (l >> 1)
                        dst_c = depth * (l & 1)
                        for k in range(depth // LANES):
                            comp_v[dst_r, pl.ds(dst_c + k * LANES, LANES)] = (
                                rows_v[st + l, pl.ds(s0 + k * LANES, LANES)])

                def cloop(kb, carry):
                    cblock(kb * LANES, kb * (LANES // 2))
                    return carry

                lax.fori_loop(0, n_full, cloop, 0)
                if tail is not None:
                    cblock(tail, tail // 2)

            def octet(o, carry):
                b0 = ebase + o * OCT
                pltpu.sync_copy(idx_hbm.at[pl.ds(b0, OCT)], idx_v)
                cps = {0: fire(0, 0)}
                for e in range(OCT):
                    rslot = e % 2
                    cslot = (e >> 1) % 2
                    if e + 1 < OCT:
                        cps[e + 1] = fire(e + 1, 1 - rslot)
                    for c in cps.pop(e):
                        c.wait()
                    if e % 2 == 0:
                        @pl.when(jnp.logical_or(o > 0, e >= 4))
                        def _():
                            pltpu.make_async_copy(
                                comp[cslot],
                                out_hbm.at[pl.ds((b0 + e) * hist2, hist)],
                                sw[cslot]).wait()

                    compact(e, rslot, cslot)
                    if e % 2 == 1:
                        pltpu.async_copy(
                            comp[cslot],
                            out_hbm.at[pl.ds((b0 + e - 1) * hist2, hist)],
                            sw[cslot])
                return carry

            lax.fori_loop(0, n_oct, octet, 0)
            pltpu.make_async_copy(
                comp0, out_hbm.at[pl.ds(ebase * hist2, hist)], sw0).wait()
            pltpu.make_async_copy(
                comp1, out_hbm.at[pl.ds(ebase * hist2, hist)], sw1).wait()

        run_table(qi_hbm, qt_hbm, qo_hbm)
        run_table(vi_hbm, kt_hbm, vo_hbm)

    return gather2


def kernel(queries, values, query_table, key_table):
    batch, hist = queries.shape
    n_rows, depth = query_table.shape
    qi = queries.astype(jnp.int32)
    vi = values.astype(jnp.int32)
    qt2 = query_table.reshape(n_rows // 2, 2 * depth)
    kt2 = key_table.reshape(n_rows // 2, 2 * depth)
    q_out, v_out = _make_gather2(batch, hist, depth)(qi, vi, qt2, kt2)
    return (q_out.reshape(batch, hist, depth),
            v_out.reshape(batch, hist, depth))


# speedup vs baseline: 1.3883x; 1.3883x over previous
import functools

import jax
import jax.numpy as jnp
from jax import lax
from jax.experimental import pallas as pl
from jax.experimental.pallas import tpu as pltpu
from jax.experimental.pallas import tpu_sc as plsc

LANES = 16
OCT = 8


@functools.cache
def _make_gather(batch, hist, depth):
    d2 = 2 * depth
    info = plsc.get_sparse_core_info()
    nw = info.num_cores * info.num_subcores
    entries_per_worker = batch // nw
    n_oct = entries_per_worker // OCT
    n_full = hist // LANES
    tail = hist - LANES if hist % LANES else None
    mesh = plsc.VectorSubcoreMesh(core_axis_name="c", subcore_axis_name="s")

    @functools.partial(
        pl.kernel,
        mesh=mesh,
        compiler_params=pltpu.CompilerParams(use_tc_tiling_on_sc=True),
        out_type=jax.ShapeDtypeStruct((batch, hist, depth), jnp.float32),
        scratch_types=[
            pltpu.VMEM((OCT, hist), jnp.int32),
            pltpu.VMEM((OCT, hist), jnp.int32),
            pltpu.VMEM((hist, d2), jnp.float32),
            pltpu.VMEM((hist, d2), jnp.float32),
            pltpu.VMEM((hist, depth), jnp.float32),
            pltpu.VMEM((hist, depth), jnp.float32),
            pltpu.SemaphoreType.DMA,
            pltpu.SemaphoreType.DMA,
            pltpu.SemaphoreType.DMA,
            pltpu.SemaphoreType.DMA,
        ],
    )
    def gather1(idx_hbm, tab_hbm, out_hbm,
                idx_v, half_v, rows0, rows1, comp0, comp1,
                sg0, sg1, sw0, sw1):
        wid = lax.axis_index("s") * info.num_cores + lax.axis_index("c")
        ebase = wid * entries_per_worker
        rows = (rows0, rows1)
        comp = (comp0, comp1)
        sg = (sg0, sg1)
        sw = (sw0, sw1)

        def fire(e, slot):
            for kb in range(n_full):
                sl = pl.ds(kb * LANES, LANES)
                half_v[e, sl] = lax.shift_right_logical(idx_v[e, sl], 1)
            if tail is not None:
                sl = pl.ds(tail, LANES)
                half_v[e, sl] = lax.shift_right_logical(idx_v[e, sl], 1)
            return [
                pltpu.async_copy(
                    tab_hbm.at[half_v.at[e, pl.ds(0, 128)]],
                    rows[slot].at[pl.ds(0, 128)],
                    sg[slot],
                ),
                pltpu.async_copy(
                    tab_hbm.at[half_v.at[e, pl.ds(128, hist - 128)]],
                    rows[slot].at[pl.ds(128, hist - 128)],
                    sg[slot],
                ),
            ]

        def compact(e, slot):
            rows_v, comp_v = rows[slot], comp[slot]

            def cblock(st):
                starts = (idx_v[e, pl.ds(st, LANES)] & 1) * depth
                for l in range(LANES):
                    s0 = starts[l]
                    for k in range(depth // LANES):
                        comp_v[st + l, pl.ds(k * LANES, LANES)] = (
                            rows_v[st + l, pl.ds(s0 + k * LANES, LANES)])

            def cloop(kb, carry):
                cblock(kb * LANES)
                return carry

            lax.fori_loop(0, n_full, cloop, 0)
            if tail is not None:
                cblock(tail)

        def octet(o, carry):
            b0 = ebase + o * OCT
            pltpu.sync_copy(idx_hbm.at[pl.ds(b0, OCT)], idx_v)
            cps = {0: fire(0, 0)}
            for e in range(OCT):
                slot = e % 2
                if e + 1 < OCT:
                    cps[e + 1] = fire(e + 1, 1 - slot)
                for c in cps.pop(e):
                    c.wait()
                @pl.when(jnp.logical_or(o > 0, e >= 2))
                def _():
                    pltpu.make_async_copy(
                        comp[slot], out_hbm.at[b0 + e], sw[slot]).wait()

                compact(e, slot)
                pltpu.async_copy(comp[slot], out_hbm.at[b0 + e], sw[slot])
            return carry

        lax.fori_loop(0, n_oct, octet, 0)
        pltpu.make_async_copy(comp0, out_hbm.at[ebase], sw0).wait()
        pltpu.make_async_copy(comp1, out_hbm.at[ebase], sw1).wait()

    return gather1


def kernel(queries, values, query_table, key_table):
    batch, hist = queries.shape
    n_rows, depth = query_table.shape
    gather = _make_gather(batch, hist, depth)
    q_out = gather(queries.astype(jnp.int32),
                   query_table.reshape(n_rows // 2, 2 * depth))
    v_out = gather(values.astype(jnp.int32),
                   key_table.reshape(n_rows // 2, 2 * depth))
    return q_out, v_out


# speedup vs baseline: 1.4003x; 1.0086x over previous
import functools

import jax
import jax.numpy as jnp
from jax import lax
from jax.experimental import pallas as pl
from jax.experimental.pallas import tpu as pltpu
from jax.experimental.pallas import tpu_sc as plsc

LANES = 16
OCT = 16


@functools.cache
def _make_gather(batch, hist, depth):
    d2 = 2 * depth
    info = plsc.get_sparse_core_info()
    nw = info.num_cores * info.num_subcores
    entries_per_worker = batch // nw
    n_oct = entries_per_worker // OCT
    n_full = hist // LANES
    tail = hist - LANES if hist % LANES else None
    mesh = plsc.VectorSubcoreMesh(core_axis_name="c", subcore_axis_name="s")

    @functools.partial(
        pl.kernel,
        mesh=mesh,
        compiler_params=pltpu.CompilerParams(use_tc_tiling_on_sc=True),
        out_type=jax.ShapeDtypeStruct((batch, hist, depth), jnp.float32),
        scratch_types=[
            pltpu.VMEM((OCT, hist), jnp.int32),
            pltpu.VMEM((OCT, hist), jnp.int32),
            pltpu.VMEM((hist, d2), jnp.float32),
            pltpu.VMEM((hist, d2), jnp.float32),
            pltpu.VMEM((hist, depth), jnp.float32),
            pltpu.VMEM((hist, depth), jnp.float32),
            pltpu.SemaphoreType.DMA,
            pltpu.SemaphoreType.DMA,
            pltpu.SemaphoreType.DMA,
            pltpu.SemaphoreType.DMA,
        ],
    )
    def gather1(idx_hbm, tab_hbm, out_hbm,
                idx_v, half_v, rows0, rows1, comp0, comp1,
                sg0, sg1, sw0, sw1):
        wid = lax.axis_index("s") * info.num_cores + lax.axis_index("c")
        ebase = wid * entries_per_worker
        rows = (rows0, rows1)
        comp = (comp0, comp1)
        sg = (sg0, sg1)
        sw = (sw0, sw1)

        def fire(e, slot):
            for kb in range(n_full):
                sl = pl.ds(kb * LANES, LANES)
                half_v[e, sl] = lax.shift_right_logical(idx_v[e, sl], 1)
            if tail is not None:
                sl = pl.ds(tail, LANES)
                half_v[e, sl] = lax.shift_right_logical(idx_v[e, sl], 1)
            return [
                pltpu.async_copy(
                    tab_hbm.at[half_v.at[e, pl.ds(0, 128)]],
                    rows[slot].at[pl.ds(0, 128)],
                    sg[slot],
                ),
                pltpu.async_copy(
                    tab_hbm.at[half_v.at[e, pl.ds(128, hist - 128)]],
                    rows[slot].at[pl.ds(128, hist - 128)],
                    sg[slot],
                ),
            ]

        def compact(e, slot):
            rows_v, comp_v = rows[slot], comp[slot]

            def cblock(st):
                starts = (idx_v[e, pl.ds(st, LANES)] & 1) * depth
                for l in range(LANES):
                    s0 = starts[l]
                    for k in range(depth // LANES):
                        comp_v[st + l, pl.ds(k * LANES, LANES)] = (
                            rows_v[st + l, pl.ds(s0 + k * LANES, LANES)])

            def cloop(kb, carry):
                cblock(kb * LANES)
                return carry

            lax.fori_loop(0, n_full, cloop, 0)
            if tail is not None:
                cblock(tail)

        def octet(o, carry):
            b0 = ebase + o * OCT
            pltpu.sync_copy(idx_hbm.at[pl.ds(b0, OCT)], idx_v)
            cps = {0: fire(0, 0)}
            for e in range(OCT):
                slot = e % 2
                if e + 1 < OCT:
                    cps[e + 1] = fire(e + 1, 1 - slot)
                for c in cps.pop(e):
                    c.wait()
                @pl.when(jnp.logical_or(o > 0, e >= 2))
                def _():
                    pltpu.make_async_copy(
                        comp[slot], out_hbm.at[b0 + e], sw[slot]).wait()

                compact(e, slot)
                pltpu.async_copy(comp[slot], out_hbm.at[b0 + e], sw[slot])
            return carry

        lax.fori_loop(0, n_oct, octet, 0)
        pltpu.make_async_copy(comp0, out_hbm.at[ebase], sw0).wait()
        pltpu.make_async_copy(comp1, out_hbm.at[ebase], sw1).wait()

    return gather1


def kernel(queries, values, query_table, key_table):
    batch, hist = queries.shape
    n_rows, depth = query_table.shape
    gather = _make_gather(batch, hist, depth)
    q_out = gather(queries.astype(jnp.int32),
                   query_table.reshape(n_rows // 2, 2 * depth))
    v_out = gather(values.astype(jnp.int32),
                   key_table.reshape(n_rows // 2, 2 * depth))
    return q_out, v_out
